# baseline (speedup 1.0000x reference)
import jax
import jax.numpy as jnp
import numpy as np
from jax import lax
from jax.experimental import pallas as pl
from jax.experimental.pallas import tpu as pltpu
from jax.experimental.pallas import tpu_sc as plsc

N = 10000
E = 160000
DF = 256
H = 256
C = 64
T = 8

NC = 2
NS = 16
LANES = 16
N2 = N // NC
WZ = H // 2

K = 64
NCHUNK = 82
EPT = NCHUNK * K
EPC = EPT * NS
RPT = 312
RREM = N2 - NS * RPT

ROWB = 1000
GRID = N // ROWB


def _zperm():
    g = np.empty((H,), dtype=np.int32)
    for w in range(WZ):
        a = 32 * (w // 16) + (w % 16)
        g[w] = a
        g[WZ + w] = a + 16
    return g


_ZPERM = _zperm()


def _linT(h, w_ref, b_ref):
    return lax.dot_general(h, w_ref[...], (((1,), (1,)), ((), ())),
                           preferred_element_type=jnp.float32) + b_ref[...]


def _pack_z(z):
    zb = lax.bitcast_convert_type(z.astype(jnp.bfloat16), jnp.uint16)
    lo = zb[:, :WZ].astype(jnp.int32)
    hi = zb[:, WZ:].astype(jnp.int32)
    return lo | (hi << 16)


def _mlp_body(x_ref, w1_ref, b1_ref, wl0_ref, bl0_ref, wl1_ref, bl1_ref,
              ww_ref, bw_ref, out_ref, z_ref):
    h = jnp.maximum(_linT(x_ref[...], w1_ref, b1_ref), 0.0)
    h = jnp.maximum(_linT(h, wl0_ref, bl0_ref), 0.0)
    h = jnp.maximum(_linT(h, wl1_ref, bl1_ref), 0.0)
    out_ref[...] = h
    z_ref[...] = _pack_z(_linT(h, ww_ref, bw_ref))


_mlp = pl.pallas_call(
    _mlp_body,
    grid=(GRID,),
    in_specs=[
        pl.BlockSpec((ROWB, DF), lambda i: (i, 0)),
        pl.BlockSpec((H, DF), lambda i: (0, 0)),
        pl.BlockSpec((1, H), lambda i: (0, 0)),
        pl.BlockSpec((H, H), lambda i: (0, 0)),
        pl.BlockSpec((1, H), lambda i: (0, 0)),
        pl.BlockSpec((H, H), lambda i: (0, 0)),
        pl.BlockSpec((1, H), lambda i: (0, 0)),
        pl.BlockSpec((H, H), lambda i: (0, 0)),
        pl.BlockSpec((1, H), lambda i: (0, 0)),
    ],
    out_specs=[
        pl.BlockSpec((ROWB, H), lambda i: (i, 0)),
        pl.BlockSpec((ROWB, WZ), lambda i: (i, 0)),
    ],
    out_shape=[
        jax.ShapeDtypeStruct((N, H), jnp.float32),
        jax.ShapeDtypeStruct((N, WZ), jnp.int32),
    ],
)


def _step_body(s_ref, o_in_ref, p_ref, ww_ref, bw_ref, out_ref, z_ref):
    s = s_ref[0]
    p = jnp.concatenate([p_ref[0], p_ref[1]], axis=1)
    o = o_in_ref[...] + s * jnp.maximum(p, 0.0)
    out_ref[...] = o
    z_ref[...] = _pack_z(_linT(o, ww_ref, bw_ref))


_step = pl.pallas_call(
    _step_body,
    grid=(GRID,),
    in_specs=[
        pl.BlockSpec(memory_space=pltpu.SMEM),
        pl.BlockSpec((ROWB, H), lambda i: (i, 0)),
        pl.BlockSpec((2, ROWB, WZ), lambda i: (0, i, 0)),
        pl.BlockSpec((H, H), lambda i: (0, 0)),
        pl.BlockSpec((1, H), lambda i: (0, 0)),
    ],
    out_specs=[
        pl.BlockSpec((ROWB, H), lambda i: (i, 0)),
        pl.BlockSpec((ROWB, WZ), lambda i: (i, 0)),
    ],
    out_shape=[
        jax.ShapeDtypeStruct((N, H), jnp.float32),
        jax.ShapeDtypeStruct((N, WZ), jnp.int32),
    ],
)


def _final_body(s_ref, o_in_ref, p_ref, wl_ref, bl_ref, o_ref):
    s = s_ref[0]
    p = jnp.concatenate([p_ref[0], p_ref[1]], axis=1)
    o = o_in_ref[...] + s * jnp.maximum(p, 0.0)
    logits = _linT(o, wl_ref, bl_ref)
    m = jnp.max(logits, axis=1, keepdims=True)
    ex = jnp.exp(logits - m)
    lse = jnp.log(jnp.sum(ex, axis=1, keepdims=True))
    o_ref[...] = logits - m - lse


_final = pl.pallas_call(
    _final_body,
    grid=(GRID,),
    in_specs=[
        pl.BlockSpec(memory_space=pltpu.SMEM),
        pl.BlockSpec((ROWB, H), lambda i: (i, 0)),
        pl.BlockSpec((2, ROWB, WZ), lambda i: (0, i, 0)),
        pl.BlockSpec((C, H), lambda i: (0, 0)),
        pl.BlockSpec((1, C), lambda i: (0, 0)),
    ],
    out_specs=pl.BlockSpec((ROWB, C), lambda i: (i, 0)),
    out_shape=jax.ShapeDtypeStruct((N, C), jnp.float32),
)


def _spmm_body(z_hbm, edge_hbm, val_hbm, out_hbm, ebuf, vbuf, gbuf, sbuf,
               acc0, acc1, esem, vsem, gsem, ssem0, ssem1):
    c = lax.axis_index("c")
    s = lax.axis_index("s")

    zv = jnp.zeros((LANES,), jnp.float32)

    def zrow(r, _):
        for f in range(WZ // LANES):
            sbuf[0, 0, r, pl.ds(f * LANES, LANES)] = zv
        return 0
    lax.fori_loop(0, K, zrow, 0)

    base = s * RPT
    for acc in (acc0, acc1):
        for kk in range(RPT // K):
            pltpu.sync_copy(sbuf.at[0, 0], acc.at[pl.ds(base + kk * K, K)])
        rem = RPT % K
        if rem:
            pltpu.sync_copy(sbuf.at[0, 0, pl.ds(0, rem)],
                            acc.at[pl.ds(base + (RPT // K) * K, rem)])

        @pl.when(s == NS - 1)
        def _():
            pltpu.sync_copy(sbuf.at[0, 0, pl.ds(0, RREM)],
                            acc.at[pl.ds(NS * RPT, RREM)])
    plsc.subcore_barrier()

    def start_edges(j):
        b = j % 4
        pltpu.async_copy(edge_hbm.at[c, s, j], ebuf.at[b], esem.at[b])
        pltpu.async_copy(val_hbm.at[c, s, j], vbuf.at[b], vsem.at[b])

    def wait_edges(j):
        b = j % 4
        pltpu.make_async_copy(edge_hbm.at[c, s, j], ebuf.at[b],
                              esem.at[b]).wait()
        pltpu.make_async_copy(val_hbm.at[c, s, j], vbuf.at[b],
                              vsem.at[b]).wait()

    def start_gather(j):
        pltpu.async_copy(z_hbm.at[ebuf.at[j % 4, 0]], gbuf.at[j % 2],
                         gsem.at[j % 2])

    def wait_gather(j):
        pltpu.make_async_copy(z_hbm.at[ebuf.at[j % 4, 0]], gbuf.at[j % 2],
                              gsem.at[j % 2]).wait()

    def start_scatter(j):
        pltpu.async_copy(sbuf.at[j % 2, 0], acc0.at[ebuf.at[j % 4, 1]],
                         ssem0.at[j % 2], add=True)
        pltpu.async_copy(sbuf.at[j % 2, 1], acc1.at[ebuf.at[j % 4, 1]],
                         ssem1.at[j % 2], add=True)

    def wait_scatter(j):
        pltpu.make_async_copy(sbuf.at[j % 2, 0], acc0.at[ebuf.at[j % 4, 1]],
                              ssem0.at[j % 2]).wait()
        pltpu.make_async_copy(sbuf.at[j % 2, 1], acc1.at[ebuf.at[j % 4, 1]],
                              ssem1.at[j % 2]).wait()

    start_edges(0)
    start_edges(1)
    start_edges(2)
    wait_edges(0)
    start_gather(0)

    def chunk(j, _):
        g = j % 2
        e4 = j % 4
        wait_gather(j)

        @pl.when(j + 1 < NCHUNK)
        def _():
            wait_edges(j + 1)
            start_gather(j + 1)

        def edge_group(eg, _):
            vals16 = vbuf[e4, pl.ds(eg * LANES, LANES)]
            for el in range(LANES):
                vb = lax.gather(
                    vals16, jnp.full((LANES, 1), el, jnp.int32),
                    lax.GatherDimensionNumbers(
                        offset_dims=(), collapsed_slice_dims=(0,),
                        start_index_map=(0,)),
                    (1,), mode=lax.GatherScatterMode.PROMISE_IN_BOUNDS)
                e = eg * LANES + el
                xs = [gbuf[g, e, pl.ds(fw * LANES, LANES)]
                      for fw in range(WZ // LANES)]
                ys = []
                for x in xs:
                    lo = lax.bitcast_convert_type(x << 16, jnp.float32)
                    hi = lax.bitcast_convert_type(x & jnp.int32(-65536),
                                                  jnp.float32)
                    ys.append((lo * vb, hi * vb))
                for fw in range(WZ // LANES):
                    ya, yb = ys[fw]
                    h = fw // 4
                    fl = fw % 4
                    sbuf[g, h, e, pl.ds(fl * 2 * LANES, LANES)] = ya
                    sbuf[g, h, e, pl.ds(fl * 2 * LANES + LANES, LANES)] = yb
            return 0
        lax.fori_loop(0, K // LANES, edge_group, 0)

        start_scatter(j)

        @pl.when(j >= 1)
        def _():
            wait_scatter(j - 1)

        @pl.when(j + 3 < NCHUNK)
        def _():
            start_edges(j + 3)
        return 0
    lax.fori_loop(0, NCHUNK, chunk, 0)

    wait_scatter(NCHUNK - 1)
    plsc.subcore_barrier()
    for hh, acc in ((0, acc0), (1, acc1)):
        pltpu.sync_copy(acc.at[pl.ds(s * RPT, RPT)],
                        out_hbm.at[hh, pl.ds(c * N2 + s * RPT, RPT)])

        @pl.when(s == NS - 1)
        def _():
            pltpu.sync_copy(acc.at[pl.ds(NS * RPT, RREM)],
                            out_hbm.at[hh, pl.ds(c * N2 + NS * RPT, RREM)])


_spmm = pl.kernel(
    _spmm_body,
    out_type=jax.ShapeDtypeStruct((2, N, WZ), jnp.float32),
    mesh=plsc.VectorSubcoreMesh(core_axis_name="c", subcore_axis_name="s",
                                num_cores=NC, num_subcores=NS),
    scratch_types=[
        pltpu.VMEM((4, 2, K), jnp.int32),
        pltpu.VMEM((4, K), jnp.float32),
        pltpu.VMEM((2, K, WZ), jnp.int32),
        pltpu.VMEM((2, 2, K, WZ), jnp.float32),
        pltpu.VMEM_SHARED((N2, WZ), jnp.float32),
        pltpu.VMEM_SHARED((N2, WZ), jnp.float32),
        pltpu.SemaphoreType.DMA((4,)),
        pltpu.SemaphoreType.DMA((4,)),
        pltpu.SemaphoreType.DMA((2,)),
        pltpu.SemaphoreType.DMA((2,)),
        pltpu.SemaphoreType.DMA((2,)),
    ],
)


def kernel(x, W1, b1, Wl0, bl0, Wl1, bl1, Ww, bw, Wlast, blast, scaler,
           A_vals, edge_row, edge_col):
    b1r = b1.reshape(1, H)
    bl0r = bl0.reshape(1, H)
    bl1r = bl1.reshape(1, H)
    Wwp = Ww[_ZPERM]
    bwr = bw[_ZPERM].reshape(1, H)
    blastr = blast.reshape(1, C)

    half = (edge_row >= N2).astype(jnp.int32)
    r1 = jnp.cumsum(half)
    r0 = jnp.arange(1, E + 1, dtype=jnp.int32) - r1
    pos = jnp.where(half == 0, r0 - 1, EPC + r1 - 1)
    colp = jnp.zeros((2 * EPC,), jnp.int32).at[pos].set(edge_col)
    rowl = jnp.zeros((2 * EPC,), jnp.int32).at[pos].set(edge_row - half * N2)
    valp = jnp.zeros((2 * EPC,), jnp.float32).at[pos].set(A_vals)
    colp = colp.reshape(NC, NS, NCHUNK, K)
    rowl = rowl.reshape(NC, NS, NCHUNK, K)
    edges = jnp.stack([colp, rowl], axis=3)
    vals = valp.reshape(NC, NS, NCHUNK, K)

    out, z = _mlp(x, W1, b1r, Wl0, bl0r, Wl1, bl1r, Wwp, bwr)
    for t in range(T):
        p = _spmm(z, edges, vals)
        st = scaler[t]
        if t < T - 1:
            out, z = _step(st, out, p, Wwp, bwr)
        else:
            res = _final(st, out, p, Wlast, blastr)
    return res


# speedup vs baseline: 3.3950x; 3.3950x over previous
import jax
import jax.numpy as jnp
from jax import lax
from jax.experimental import pallas as pl
from jax.experimental.pallas import tpu as pltpu
from jax.experimental.pallas import tpu_sc as plsc

N = 10000
E = 160000
DF = 256
H = 256
C = 64
T = 8

NC = 2
NS = 16
LANES = 16
HH = H // NC

K = 80
NCHUNK = 125
EPT = NCHUNK * K
EPAD = EPT * NS
RPT = 624
RREM = N - NS * RPT

ROWB = 1000
GRID = N // ROWB


def _linT(h, w_ref, b_ref):
    return lax.dot_general(h, w_ref[...], (((1,), (1,)), ((), ())),
                           preferred_element_type=jnp.float32) + b_ref[...]


def _mlp_body(x_ref, w1_ref, b1_ref, wl0_ref, bl0_ref, wl1_ref, bl1_ref,
              ww_ref, bw_ref, out_ref, z_ref):
    h = jnp.maximum(_linT(x_ref[...], w1_ref, b1_ref), 0.0)
    h = jnp.maximum(_linT(h, wl0_ref, bl0_ref), 0.0)
    h = jnp.maximum(_linT(h, wl1_ref, bl1_ref), 0.0)
    out_ref[...] = h
    z = _linT(h, ww_ref, bw_ref)
    z_ref[0] = z[:, :HH]
    z_ref[1] = z[:, HH:]


_mlp = pl.pallas_call(
    _mlp_body,
    grid=(GRID,),
    in_specs=[
        pl.BlockSpec((ROWB, DF), lambda i: (i, 0)),
        pl.BlockSpec((H, DF), lambda i: (0, 0)),
        pl.BlockSpec((1, H), lambda i: (0, 0)),
        pl.BlockSpec((H, H), lambda i: (0, 0)),
        pl.BlockSpec((1, H), lambda i: (0, 0)),
        pl.BlockSpec((H, H), lambda i: (0, 0)),
        pl.BlockSpec((1, H), lambda i: (0, 0)),
        pl.BlockSpec((H, H), lambda i: (0, 0)),
        pl.BlockSpec((1, H), lambda i: (0, 0)),
    ],
    out_specs=[
        pl.BlockSpec((ROWB, H), lambda i: (i, 0)),
        pl.BlockSpec((2, ROWB, HH), lambda i: (0, i, 0)),
    ],
    out_shape=[
        jax.ShapeDtypeStruct((N, H), jnp.float32),
        jax.ShapeDtypeStruct((2, N, HH), jnp.float32),
    ],
)


def _step_body(s_ref, o_in_ref, p_ref, ww_ref, bw_ref, out_ref, z_ref):
    s = s_ref[0]
    p = jnp.concatenate([p_ref[0], p_ref[1]], axis=1)
    o = o_in_ref[...] + s * jnp.maximum(p, 0.0)
    out_ref[...] = o
    z = _linT(o, ww_ref, bw_ref)
    z_ref[0] = z[:, :HH]
    z_ref[1] = z[:, HH:]


_step = pl.pallas_call(
    _step_body,
    grid=(GRID,),
    in_specs=[
        pl.BlockSpec(memory_space=pltpu.SMEM),
        pl.BlockSpec((ROWB, H), lambda i: (i, 0)),
        pl.BlockSpec((2, ROWB, HH), lambda i: (0, i, 0)),
        pl.BlockSpec((H, H), lambda i: (0, 0)),
        pl.BlockSpec((1, H), lambda i: (0, 0)),
    ],
    out_specs=[
        pl.BlockSpec((ROWB, H), lambda i: (i, 0)),
        pl.BlockSpec((2, ROWB, HH), lambda i: (0, i, 0)),
    ],
    out_shape=[
        jax.ShapeDtypeStruct((N, H), jnp.float32),
        jax.ShapeDtypeStruct((2, N, HH), jnp.float32),
    ],
)


def _final_body(s_ref, o_in_ref, p_ref, wl_ref, bl_ref, o_ref):
    s = s_ref[0]
    p = jnp.concatenate([p_ref[0], p_ref[1]], axis=1)
    o = o_in_ref[...] + s * jnp.maximum(p, 0.0)
    logits = _linT(o, wl_ref, bl_ref)
    m = jnp.max(logits, axis=1, keepdims=True)
    ex = jnp.exp(logits - m)
    lse = jnp.log(jnp.sum(ex, axis=1, keepdims=True))
    o_ref[...] = logits - m - lse


_final = pl.pallas_call(
    _final_body,
    grid=(GRID,),
    in_specs=[
        pl.BlockSpec(memory_space=pltpu.SMEM),
        pl.BlockSpec((ROWB, H), lambda i: (i, 0)),
        pl.BlockSpec((2, ROWB, HH), lambda i: (0, i, 0)),
        pl.BlockSpec((C, H), lambda i: (0, 0)),
        pl.BlockSpec((1, C), lambda i: (0, 0)),
    ],
    out_specs=pl.BlockSpec((ROWB, C), lambda i: (i, 0)),
    out_shape=jax.ShapeDtypeStruct((N, C), jnp.float32),
)


def _spmm_body(z_hbm, edge_hbm, val_hbm, out_hbm, ebuf, vbuf, gbuf, sbuf,
               acc, esem, vsem, gsem, ssem):
    c = lax.axis_index("c")
    s = lax.axis_index("s")

    zv = jnp.zeros((LANES,), jnp.float32)

    def zrow(r, _):
        for f in range(HH // LANES):
            gbuf[0, r, pl.ds(f * LANES, LANES)] = zv
        return 0
    lax.fori_loop(0, K, zrow, 0)

    base = s * RPT
    for kk in range(RPT // K):
        pltpu.sync_copy(gbuf.at[0], acc.at[pl.ds(base + kk * K, K)])
    rem = RPT % K
    if rem:
        pltpu.sync_copy(gbuf.at[0, pl.ds(0, rem)],
                        acc.at[pl.ds(base + (RPT // K) * K, rem)])

    @pl.when(s == NS - 1)
    def _():
        pltpu.sync_copy(gbuf.at[0, pl.ds(0, RREM)],
                        acc.at[pl.ds(NS * RPT, RREM)])
    plsc.subcore_barrier()

    def start_edges(j):
        b = j % 4
        pltpu.async_copy(edge_hbm.at[s, j], ebuf.at[b], esem.at[b])
        pltpu.async_copy(val_hbm.at[s, j], vbuf.at[b], vsem.at[b])

    def wait_edges(j):
        b = j % 4
        pltpu.make_async_copy(edge_hbm.at[s, j], ebuf.at[b],
                              esem.at[b]).wait()
        pltpu.make_async_copy(val_hbm.at[s, j], vbuf.at[b],
                              vsem.at[b]).wait()

    def start_gather(j):
        pltpu.async_copy(z_hbm.at[ebuf.at[j % 4, c]], gbuf.at[j % 2],
                         gsem.at[j % 2])

    def wait_gather(j):
        pltpu.make_async_copy(z_hbm.at[ebuf.at[j % 4, c]], gbuf.at[j % 2],
                              gsem.at[j % 2]).wait()

    def start_scatter(j):
        pltpu.async_copy(sbuf.at[j % 2], acc.at[ebuf.at[j % 4, 2]],
                         ssem.at[j % 2], add=True)

    def wait_scatter(j):
        pltpu.make_async_copy(sbuf.at[j % 2], acc.at[ebuf.at[j % 4, 2]],
                              ssem.at[j % 2]).wait()

    start_edges(0)
    start_edges(1)
    start_edges(2)
    wait_edges(0)
    start_gather(0)

    def chunk(j, _):
        g = j % 2
        e4 = j % 4
        wait_gather(j)

        @pl.when(j + 1 < NCHUNK)
        def _():
            wait_edges(j + 1)
            start_gather(j + 1)

        def edge_group(eg, _):
            vals16 = vbuf[e4, pl.ds(eg * LANES, LANES)]
            for el in range(LANES):
                vb = lax.gather(
                    vals16, jnp.full((LANES, 1), el, jnp.int32),
                    lax.GatherDimensionNumbers(
                        offset_dims=(), collapsed_slice_dims=(0,),
                        start_index_map=(0,)),
                    (1,), mode=lax.GatherScatterMode.PROMISE_IN_BOUNDS)
                e = eg * LANES + el
                xs = [gbuf[g, e, pl.ds(f * LANES, LANES)]
                      for f in range(HH // LANES)]
                ys = [x * vb for x in xs]
                for f in range(HH // LANES):
                    sbuf[g, e, pl.ds(f * LANES, LANES)] = ys[f]
            return 0
        lax.fori_loop(0, K // LANES, edge_group, 0)

        start_scatter(j)

        @pl.when(j >= 1)
        def _():
            wait_scatter(j - 1)

        @pl.when(j + 3 < NCHUNK)
        def _():
            start_edges(j + 3)
        return 0
    lax.fori_loop(0, NCHUNK, chunk, 0)

    wait_scatter(NCHUNK - 1)
    plsc.subcore_barrier()
    pltpu.sync_copy(acc.at[pl.ds(s * RPT, RPT)],
                    out_hbm.at[pl.ds(c * N + s * RPT, RPT)])

    @pl.when(s == NS - 1)
    def _():
        pltpu.sync_copy(acc.at[pl.ds(NS * RPT, RREM)],
                        out_hbm.at[pl.ds(c * N + NS * RPT, RREM)])


_spmm = pl.kernel(
    _spmm_body,
    out_type=jax.ShapeDtypeStruct((2 * N, HH), jnp.float32),
    mesh=plsc.VectorSubcoreMesh(core_axis_name="c", subcore_axis_name="s",
                                num_cores=NC, num_subcores=NS),
    scratch_types=[
        pltpu.VMEM((4, 3, K), jnp.int32),
        pltpu.VMEM((4, K), jnp.float32),
        pltpu.VMEM((2, K, HH), jnp.float32),
        pltpu.VMEM((2, K, HH), jnp.float32),
        pltpu.VMEM_SHARED((N, HH), jnp.float32),
        pltpu.SemaphoreType.DMA((4,)),
        pltpu.SemaphoreType.DMA((4,)),
        pltpu.SemaphoreType.DMA((2,)),
        pltpu.SemaphoreType.DMA((2,)),
    ],
)


def kernel(x, W1, b1, Wl0, bl0, Wl1, bl1, Ww, bw, Wlast, blast, scaler,
           A_vals, edge_row, edge_col):
    b1r = b1.reshape(1, H)
    bl0r = bl0.reshape(1, H)
    bl1r = bl1.reshape(1, H)
    bwr = bw.reshape(1, H)
    blastr = blast.reshape(1, C)

    colp = edge_col.reshape(NS, NCHUNK, K)
    rowp = edge_row.reshape(NS, NCHUNK, K)
    valp = A_vals.reshape(NS, NCHUNK, K)
    edges = jnp.stack([colp, colp + N, rowp], axis=2)

    out, z = _mlp(x, W1, b1r, Wl0, bl0r, Wl1, bl1r, Ww, bwr)
    zf = z.reshape(2 * N, HH)
    for t in range(T):
        p = _spmm(zf, edges, valp)
        st = scaler[t]
        if t < T - 1:
            out, z = _step(st, out, p.reshape(2, N, HH), Ww, bwr)
            zf = z.reshape(2 * N, HH)
        else:
            res = _final(st, out, p.reshape(2, N, HH), Wlast, blastr)
    return res
